# baseline (speedup 1.0000x reference)
import functools

import jax
import jax.numpy as jnp
from jax import lax
from jax.experimental import pallas as pl
from jax.experimental.pallas import tpu as pltpu
from jax.experimental.pallas import tpu_sc as plsc

_D = 128
_CH = 256


def _make_sc_kernel(n_img, positions):
    info = plsc.get_sparse_core_info()
    NC, NS = info.num_cores, info.num_subcores
    NW = NC * NS
    B = n_img * positions
    b_per_w = B // NW
    n_chunks = b_per_w // _CH
    w_per_img = positions // b_per_w
    assert b_per_w % _CH == 0 and positions % b_per_w == 0
    mesh = plsc.VectorSubcoreMesh(core_axis_name="c", subcore_axis_name="s")

    @functools.partial(
        pl.kernel, mesh=mesh,
        out_type=jax.ShapeDtypeStruct((n_img, _D, positions), jnp.float32),
        compiler_params=pltpu.CompilerParams(needs_layout_passes=False),
        scratch_types=[
            pltpu.VMEM((_CH,), jnp.int32),
            pltpu.VMEM((_CH, _D), jnp.float32),
            pltpu.VMEM((_D, _CH), jnp.float32),
            pltpu.SemaphoreType.DMA,
        ],
    )
    def k(idx_hbm, table_hbm, out_hbm, idx_v, rows_v, cols_v, sem):
        wid = lax.axis_index("s") * NC + lax.axis_index("c")
        nimg = wid // w_per_img
        pbase = (wid % w_per_img) * b_per_w
        lane = lax.broadcasted_iota(jnp.int32, (16,), 0)

        def body(i, carry):
            poff = pbase + i * _CH
            pltpu.sync_copy(idx_hbm.at[pl.ds(wid * b_per_w + i * _CH, _CH)], idx_v)
            pltpu.async_copy(table_hbm.at[idx_v], rows_v, sem).wait()

            def tr_body(c, carry2):
                cvec = jnp.full((16,), c, jnp.int32)
                for p0 in range(0, _CH, 16):
                    v = plsc.load_gather(rows_v, [p0 + lane, cvec])
                    cols_v[c, pl.ds(p0, 16)] = v + 1.0
                return carry2

            lax.fori_loop(0, _D, tr_body, 0)
            pltpu.sync_copy(cols_v, out_hbm.at[nimg, :, pl.ds(poff, _CH)])
            return carry

        lax.fori_loop(0, n_chunks, body, 0)

    return k


def kernel(x, pe):
    n, _, h, w = x.shape
    positions = h * w
    idx = x.reshape(n * positions)
    out = _make_sc_kernel(n, positions)(idx, pe)
    return out.reshape(n, _D, h, w)


# speedup vs baseline: 1.6631x; 1.6631x over previous
import functools

import jax
import jax.numpy as jnp
from jax import lax
from jax.experimental import pallas as pl
from jax.experimental.pallas import tpu as pltpu
from jax.experimental.pallas import tpu_sc as plsc

_D = 128
_CH = 256


def _make_sc_kernel(n_img, positions):
    info = plsc.get_sparse_core_info()
    NC, NS = info.num_cores, info.num_subcores
    NW = NC * NS
    B = n_img * positions
    b_per_w = B // NW
    n_chunks = b_per_w // _CH
    w_per_img = positions // b_per_w
    assert b_per_w % _CH == 0 and positions % b_per_w == 0
    mesh = plsc.VectorSubcoreMesh(core_axis_name="c", subcore_axis_name="s")

    @functools.partial(
        pl.kernel, mesh=mesh,
        out_type=jax.ShapeDtypeStruct((n_img, _D, positions), jnp.float32),
        compiler_params=pltpu.CompilerParams(needs_layout_passes=False),
        scratch_types=[
            pltpu.VMEM((_CH,), jnp.int32),
            pltpu.VMEM((_CH, _D), jnp.float32),
            pltpu.VMEM((_D, _CH), jnp.float32),
            pltpu.SemaphoreType.DMA,
        ],
    )
    def k(idx_hbm, table_hbm, out_hbm, idx_v, rows_v, cols_v, sem):
        wid = lax.axis_index("s") * NC + lax.axis_index("c")
        nimg = wid // w_per_img
        pbase = (wid % w_per_img) * b_per_w
        lane = lax.broadcasted_iota(jnp.int32, (16,), 0)

        def body(i, carry):
            poff = pbase + i * _CH
            pltpu.sync_copy(idx_hbm.at[pl.ds(wid * b_per_w + i * _CH, _CH)], idx_v)
            pltpu.async_copy(table_hbm.at[idx_v], rows_v, sem).wait()

            @plsc.parallel_loop(0, _D, 1, unroll=4)
            def tr_body(c):
                cvec = jnp.full((16,), c, jnp.int32)
                for p0 in range(0, _CH, 16):
                    v = plsc.load_gather(rows_v, [p0 + lane, cvec])
                    cols_v[c, pl.ds(p0, 16)] = v + 1.0
            pltpu.sync_copy(cols_v, out_hbm.at[nimg, :, pl.ds(poff, _CH)])
            return carry

        lax.fori_loop(0, n_chunks, body, 0)

    return k


def kernel(x, pe):
    n, _, h, w = x.shape
    positions = h * w
    idx = x.reshape(n * positions)
    out = _make_sc_kernel(n, positions)(idx, pe)
    return out.reshape(n, _D, h, w)
